# baseline (speedup 1.0000x reference)
import functools

import jax
import jax.numpy as jnp
from jax import lax
from jax.experimental import pallas as pl
from jax.experimental.pallas import tpu as pltpu
from jax.experimental.pallas import tpu_sc as plsc

EPS = 1e-7
T = 0.07
NUM_CLASSES = 3
P = 5
B = 1024
FD = 128
ND = 100000
K = 2048
BQ = 256
BM = 2048
NM = 49
NDP = NM * BM
NQ = B // BQ
VALID = P + K
GW = 2176
CW = 128
DUMMY_ID = 1 << 30
NEG_BIG = -3e30
PAD_SIM = -1e30


def _embed_body(fs_ref, ft_ref, ws_ref, bs_ref, wt_ref, bt_ref,
                qr1_ref, qr2_ref, v1_ref, v2_ref, q1n_ref, q2n_ref):
    v1 = jnp.dot(fs_ref[...], ws_ref[...], preferred_element_type=jnp.float32) + bs_ref[...]
    v1_ref[...] = v1 / jnp.sqrt(jnp.sum(v1 * v1, axis=1, keepdims=True))
    v2 = jnp.dot(ft_ref[...], wt_ref[...], preferred_element_type=jnp.float32) + bt_ref[...]
    v2_ref[...] = v2 / jnp.sqrt(jnp.sum(v2 * v2, axis=1, keepdims=True))
    q1 = qr1_ref[...]
    q1n_ref[...] = q1 / (jnp.sqrt(jnp.sum(q1 * q1, axis=1, keepdims=True)) + 1e-12)
    q2 = qr2_ref[...]
    q2n_ref[...] = q2 / (jnp.sqrt(jnp.sum(q2 * q2, axis=1, keepdims=True)) + 1e-12)


def _embed(f_s, f_t, W_s, b_s, W_t, b_t, qr1, qr2):
    out = [jax.ShapeDtypeStruct((B, FD), jnp.float32)] * 4
    return pl.pallas_call(_embed_body, out_shape=out)(
        f_s, f_t, W_s, b_s.reshape(1, FD), W_t, b_t.reshape(1, FD), qr1, qr2)


def _sim_body(qn_ref, v_ref, lbl_ref, mem_ref, d_ref, tv_ref, ti_ref,
              cv_ref, ci_ref):
    mb = pl.program_id(1)
    memblk = mem_ref[...]
    dn = (((1,), (1,)), ((), ()))
    sm = lax.dot_general(qn_ref[...], memblk, dn, preferred_element_type=jnp.float32)
    d_ref[...] = lax.dot_general(v_ref[...], memblk, dn, preferred_element_type=jnp.float32)
    ns = jnp.sum(memblk * memblk, axis=1)
    rn = 1.0 / (jnp.sqrt(ns) + 1e-12)
    colrow = mb * BM + lax.broadcasted_iota(jnp.int32, (1, BM), 1)
    inclass = (colrow % NUM_CLASSES) == lbl_ref[...]
    fill = jnp.where(colrow < ND, 0.0, PAD_SIM)
    sim = jnp.where(inclass, sm * rn[None, :],
                    jnp.broadcast_to(fill, (BQ, BM)))

    lane0 = mb * BM + lax.broadcasted_iota(jnp.int32, (BQ, CW), 1)
    best = sim[:, 0:CW]
    bid = lane0
    for k in range(1, BM // CW):
        s = sim[:, k * CW:(k + 1) * CW]
        gt = s > best
        best = jnp.where(gt, s, best)
        bid = jnp.where(gt, lane0 + k * CW, bid)

    @pl.when(mb == 0)
    def _():
        cv_ref[...] = jnp.full((BQ, CW), NEG_BIG, jnp.float32)
        ci_ref[...] = DUMMY_ID + lax.broadcasted_iota(jnp.int32, (BQ, CW), 1)

    cval = jnp.concatenate([cv_ref[...], best], axis=1)
    cid = jnp.concatenate([ci_ref[...], bid], axis=1)
    vs, ids = [], []
    for _ in range(P):
        m = jnp.max(cval, axis=1, keepdims=True)
        sel = jnp.min(jnp.where(cval == m, cid, jnp.int32(0x7FFFFFFF)),
                      axis=1, keepdims=True)
        vs.append(m)
        ids.append(sel)
        cval = jnp.where(cid == sel, NEG_BIG, cval)
    newv = jnp.concatenate(vs, axis=1)
    newi = jnp.concatenate(ids, axis=1)
    cv_ref[...] = jnp.full((BQ, CW), NEG_BIG, jnp.float32)
    ci_ref[...] = DUMMY_ID + lax.broadcasted_iota(jnp.int32, (BQ, CW), 1)
    cv_ref[:, 0:P] = newv
    ci_ref[:, 0:P] = newi

    @pl.when(mb == NM - 1)
    def _():
        tv_ref[...] = jnp.concatenate(
            [newv, jnp.zeros((BQ, 8 - P), jnp.float32)], axis=1)
        ti_ref[...] = jnp.concatenate(
            [newi, jnp.zeros((BQ, 8 - P), jnp.int32)], axis=1)


def _simdots(qn, v, lbl2, memp):
    rows = qn.shape[0]
    return pl.pallas_call(
        _sim_body,
        grid=(rows // BQ, NM),
        in_specs=[
            pl.BlockSpec((BQ, FD), lambda qb, mb: (qb, 0)),
            pl.BlockSpec((BQ, FD), lambda qb, mb: (qb, 0)),
            pl.BlockSpec((BQ, 1), lambda qb, mb: (qb, 0)),
            pl.BlockSpec((BM, FD), lambda qb, mb: (mb, 0)),
        ],
        out_specs=[
            pl.BlockSpec((BQ, BM), lambda qb, mb: (qb, mb)),
            pl.BlockSpec((BQ, 8), lambda qb, mb: (qb, 0)),
            pl.BlockSpec((BQ, 8), lambda qb, mb: (qb, 0)),
        ],
        out_shape=[
            jax.ShapeDtypeStruct((rows, NDP), jnp.float32),
            jax.ShapeDtypeStruct((rows, 8), jnp.float32),
            jax.ShapeDtypeStruct((rows, 8), jnp.int32),
        ],
        scratch_shapes=[
            pltpu.VMEM((BQ, CW), jnp.float32),
            pltpu.VMEM((BQ, CW), jnp.int32),
        ],
    )(qn, v, lbl2, memp)


_PB = 128


def _prep_body(idx_ref, ti_ref, g_ref):
    pb = pl.program_id(0)
    rowoff = (pb * _PB + lax.broadcasted_iota(jnp.int32, (_PB, 1), 0)) * NDP
    g_ref[:, 0:P] = ti_ref[:, 0:P] + rowoff
    g_ref[:, P:VALID] = idx_ref[:, 1:K + 1] + rowoff
    g_ref[:, VALID:GW] = jnp.broadcast_to(rowoff, (_PB, GW - VALID))


def _prep(idx, ti):
    rows = idx.shape[0]
    return pl.pallas_call(
        _prep_body,
        grid=(rows // _PB,),
        in_specs=[
            pl.BlockSpec((_PB, K + 1), lambda pb: (pb, 0)),
            pl.BlockSpec((_PB, 8), lambda pb: (pb, 0)),
        ],
        out_specs=pl.BlockSpec((_PB, GW), lambda pb: (pb, 0)),
        out_shape=jax.ShapeDtypeStruct((rows, GW), jnp.int32),
    )(idx, ti)


_GRP = 16
_GE = _GRP * 128


def _gather_sc(d2d, giflat):
    info = plsc.get_sparse_core_info()
    nw = info.num_cores * info.num_subcores
    n_idx = giflat.shape[0]
    per_w = n_idx // nw
    ge = _GE if per_w % _GE == 0 else _GE // 2
    grp_n = ge // 128
    groups = per_w // ge

    def body(dflat, gi_hbm, out_hbm, idx_v, val_v, sem):
        wid = lax.axis_index("s") * info.num_cores + lax.axis_index("c")
        base = wid * per_w

        def grp(g, carry):
            off = base + g * ge
            pltpu.sync_copy(gi_hbm.at[pl.ds(off, ge)], idx_v)
            cps = [
                pltpu.async_copy(dflat.at[idx_v.at[pl.ds(b * 128, 128)]],
                                 val_v.at[pl.ds(b * 128, 128)], sem)
                for b in range(grp_n)
            ]
            for cp in cps:
                cp.wait()
            pltpu.sync_copy(val_v, out_hbm.at[pl.ds(off, ge)])
            return carry

        lax.fori_loop(0, groups, grp, 0)

    mesh = plsc.VectorSubcoreMesh(core_axis_name="c", subcore_axis_name="s")
    return pl.kernel(
        body,
        out_type=jax.ShapeDtypeStruct((n_idx,), jnp.float32),
        mesh=mesh,
        scratch_types=[
            pltpu.VMEM((ge,), jnp.int32),
            pltpu.VMEM((ge,), jnp.float32),
            pltpu.SemaphoreType.DMA,
        ],
    )(d2d, giflat)


def _loss_body(g1_ref, g2_ref, tv1_ref, tv2_ref, sw_ref, out_ref):
    cid = lax.broadcasted_iota(jnp.int32, (B, GW), 1)
    colmask = cid < VALID
    negmask = colmask & (cid >= P)
    c = K * (1.0 / ND)
    sw = sw_ref[...]

    def side_loss(g, tv):
        x = jnp.where(colmask, jnp.exp(g * (1.0 / T)), 0.0)
        z = (jnp.sum(x) / (B * VALID)) * ND
        out = x / z
        ppos = out[:, 0:P]
        log_d1 = jnp.log(ppos / (ppos + c + EPS))
        log_d0 = jnp.where(negmask, jnp.log(c / (out + c + EPS)), 0.0)
        s0 = jnp.sum(log_d0, axis=1, keepdims=True)
        ks = tv[:, 0:P]
        num = jnp.sum((log_d1 + s0) * ks, axis=1, keepdims=True)
        den = jnp.sum(ks, axis=1, keepdims=True)
        return jnp.sum(-(num / den) * sw, axis=(0, 1), keepdims=True) / B

    loss_s = side_loss(g2_ref[...], tv2_ref[...])
    loss_t = side_loss(g1_ref[...], tv1_ref[...])
    out_ref[...] = loss_s + loss_t


def _loss(g1, g2, tv1, tv2, sw):
    return pl.pallas_call(
        _loss_body,
        out_shape=jax.ShapeDtypeStruct((1, 1), jnp.float32),
    )(g1, g2, tv1, tv2, sw)


def kernel(f_s, f_t, sample_weights, idx, batch_label, num_pos,
           W_s, b_s, W_t, b_t, memory_v1, memory_v2):
    idx = idx.astype(jnp.int32)
    qr1 = memory_v1[idx[:, 0]]
    qr2 = memory_v2[idx[:, 0]]
    v1, v2, q1n, q2n = _embed(f_s, f_t, W_s, b_s, W_t, b_t, qr1, qr2)
    mem1p = jnp.pad(memory_v1, ((0, NDP - ND), (0, 0)))
    mem2p = jnp.pad(memory_v2, ((0, NDP - ND), (0, 0)))
    lbl2 = batch_label.astype(jnp.int32).reshape(B, 1)
    nchunk = 4
    h = B // nchunk
    chunks = []
    for qn, v, memp in ((q1n, v2, mem1p), (q2n, v1, mem2p)):
        for r0 in range(0, B, h):
            d, tv, ti = _simdots(qn[r0:r0 + h], v[r0:r0 + h],
                                 lbl2[r0:r0 + h], memp)
            gi = _prep(idx[r0:r0 + h], ti)
            g = _gather_sc(d.reshape(-1), gi.reshape(-1)).reshape(h, GW)
            chunks.append((g, tv))
    g1 = jnp.concatenate([c[0] for c in chunks[:nchunk]], axis=0)
    tv1 = jnp.concatenate([c[1] for c in chunks[:nchunk]], axis=0)
    g2 = jnp.concatenate([c[0] for c in chunks[nchunk:]], axis=0)
    tv2 = jnp.concatenate([c[1] for c in chunks[nchunk:]], axis=0)
    loss = _loss(g1, g2, tv1, tv2, sample_weights.reshape(B, 1))
    return loss.reshape(())


# speedup vs baseline: 1.0325x; 1.0325x over previous
import functools

import jax
import jax.numpy as jnp
from jax import lax
from jax.experimental import pallas as pl
from jax.experimental.pallas import tpu as pltpu
from jax.experimental.pallas import tpu_sc as plsc

EPS = 1e-7
T = 0.07
NUM_CLASSES = 3
P = 5
B = 1024
FD = 128
ND = 100000
K = 2048
BQ = 256
BM = 2048
NM = 49
NDP = NM * BM
NQ = B // BQ
VALID = P + K
GW = 2176
CW = 128
DUMMY_ID = 1 << 30
NEG_BIG = -3e30
PAD_SIM = -1e30


def _embed_body(fs_ref, ft_ref, ws_ref, bs_ref, wt_ref, bt_ref,
                qr1_ref, qr2_ref, v1_ref, v2_ref, q1n_ref, q2n_ref):
    v1 = jnp.dot(fs_ref[...], ws_ref[...], preferred_element_type=jnp.float32) + bs_ref[...]
    v1_ref[...] = v1 / jnp.sqrt(jnp.sum(v1 * v1, axis=1, keepdims=True))
    v2 = jnp.dot(ft_ref[...], wt_ref[...], preferred_element_type=jnp.float32) + bt_ref[...]
    v2_ref[...] = v2 / jnp.sqrt(jnp.sum(v2 * v2, axis=1, keepdims=True))
    q1 = qr1_ref[...]
    q1n_ref[...] = q1 / (jnp.sqrt(jnp.sum(q1 * q1, axis=1, keepdims=True)) + 1e-12)
    q2 = qr2_ref[...]
    q2n_ref[...] = q2 / (jnp.sqrt(jnp.sum(q2 * q2, axis=1, keepdims=True)) + 1e-12)


def _embed(f_s, f_t, W_s, b_s, W_t, b_t, qr1, qr2):
    out = [jax.ShapeDtypeStruct((B, FD), jnp.float32)] * 4
    return pl.pallas_call(_embed_body, out_shape=out)(
        f_s, f_t, W_s, b_s.reshape(1, FD), W_t, b_t.reshape(1, FD), qr1, qr2)


def _sim_body(qn_ref, v_ref, lbl_ref, mem_ref, d_ref, tv_ref, ti_ref,
              cv_ref, ci_ref):
    mb = pl.program_id(1)
    memblk = mem_ref[...]
    dn = (((1,), (1,)), ((), ()))
    sm = lax.dot_general(qn_ref[...], memblk, dn, preferred_element_type=jnp.float32)
    d_ref[...] = lax.dot_general(v_ref[...], memblk, dn, preferred_element_type=jnp.float32)
    ns = jnp.sum(memblk * memblk, axis=1)
    rn = 1.0 / (jnp.sqrt(ns) + 1e-12)
    colrow = mb * BM + lax.broadcasted_iota(jnp.int32, (1, BM), 1)
    inclass = (colrow % NUM_CLASSES) == lbl_ref[...]
    fill = jnp.where(colrow < ND, 0.0, PAD_SIM)
    sim = jnp.where(inclass, sm * rn[None, :],
                    jnp.broadcast_to(fill, (BQ, BM)))

    lane0 = mb * BM + lax.broadcasted_iota(jnp.int32, (BQ, CW), 1)
    best = sim[:, 0:CW]
    bid = lane0
    for k in range(1, BM // CW):
        s = sim[:, k * CW:(k + 1) * CW]
        gt = s > best
        best = jnp.where(gt, s, best)
        bid = jnp.where(gt, lane0 + k * CW, bid)

    @pl.when(mb == 0)
    def _():
        cv_ref[...] = jnp.full((BQ, CW), NEG_BIG, jnp.float32)
        ci_ref[...] = DUMMY_ID + lax.broadcasted_iota(jnp.int32, (BQ, CW), 1)

    cval = jnp.concatenate([cv_ref[...], best], axis=1)
    cid = jnp.concatenate([ci_ref[...], bid], axis=1)
    vs, ids = [], []
    for _ in range(P):
        m = jnp.max(cval, axis=1, keepdims=True)
        sel = jnp.min(jnp.where(cval == m, cid, jnp.int32(0x7FFFFFFF)),
                      axis=1, keepdims=True)
        vs.append(m)
        ids.append(sel)
        cval = jnp.where(cid == sel, NEG_BIG, cval)
    newv = jnp.concatenate(vs, axis=1)
    newi = jnp.concatenate(ids, axis=1)
    cv_ref[...] = jnp.full((BQ, CW), NEG_BIG, jnp.float32)
    ci_ref[...] = DUMMY_ID + lax.broadcasted_iota(jnp.int32, (BQ, CW), 1)
    cv_ref[:, 0:P] = newv
    ci_ref[:, 0:P] = newi

    @pl.when(mb == NM - 1)
    def _():
        tv_ref[...] = jnp.concatenate(
            [newv, jnp.zeros((BQ, 8 - P), jnp.float32)], axis=1)
        ti_ref[...] = jnp.concatenate(
            [newi, jnp.zeros((BQ, 8 - P), jnp.int32)], axis=1)


def _simdots(qn, v, lbl2, memp):
    rows = qn.shape[0]
    return pl.pallas_call(
        _sim_body,
        grid=(rows // BQ, NM),
        in_specs=[
            pl.BlockSpec((BQ, FD), lambda qb, mb: (qb, 0)),
            pl.BlockSpec((BQ, FD), lambda qb, mb: (qb, 0)),
            pl.BlockSpec((BQ, 1), lambda qb, mb: (qb, 0)),
            pl.BlockSpec((BM, FD), lambda qb, mb: (mb, 0)),
        ],
        out_specs=[
            pl.BlockSpec((BQ, BM), lambda qb, mb: (qb, mb)),
            pl.BlockSpec((BQ, 8), lambda qb, mb: (qb, 0)),
            pl.BlockSpec((BQ, 8), lambda qb, mb: (qb, 0)),
        ],
        out_shape=[
            jax.ShapeDtypeStruct((rows, NDP), jnp.float32),
            jax.ShapeDtypeStruct((rows, 8), jnp.float32),
            jax.ShapeDtypeStruct((rows, 8), jnp.int32),
        ],
        scratch_shapes=[
            pltpu.VMEM((BQ, CW), jnp.float32),
            pltpu.VMEM((BQ, CW), jnp.int32),
        ],
    )(qn, v, lbl2, memp)


_PB = 128


def _prep_body(idx_ref, ti_ref, g_ref):
    pb = pl.program_id(0)
    rowoff = (pb * _PB + lax.broadcasted_iota(jnp.int32, (_PB, 1), 0)) * NDP
    g_ref[:, 0:P] = ti_ref[:, 0:P] + rowoff
    g_ref[:, P:VALID] = idx_ref[:, 1:K + 1] + rowoff
    g_ref[:, VALID:GW] = jnp.broadcast_to(rowoff, (_PB, GW - VALID))


def _prep(idx, ti):
    rows = idx.shape[0]
    return pl.pallas_call(
        _prep_body,
        grid=(rows // _PB,),
        in_specs=[
            pl.BlockSpec((_PB, K + 1), lambda pb: (pb, 0)),
            pl.BlockSpec((_PB, 8), lambda pb: (pb, 0)),
        ],
        out_specs=pl.BlockSpec((_PB, GW), lambda pb: (pb, 0)),
        out_shape=jax.ShapeDtypeStruct((rows, GW), jnp.int32),
    )(idx, ti)


_GRP = 16
_GE = _GRP * 128


def _gather_sc(d2d, giflat):
    info = plsc.get_sparse_core_info()
    nw = info.num_cores * info.num_subcores
    n_idx = giflat.shape[0]
    per_w = n_idx // nw
    ge = _GE if per_w % _GE == 0 else _GE // 2
    grp_n = ge // 128
    groups = per_w // ge

    def body(dflat, gi_hbm, out_hbm, idx_v, val_v, sem):
        wid = lax.axis_index("s") * info.num_cores + lax.axis_index("c")
        base = wid * per_w

        def grp(g, carry):
            off = base + g * ge
            pltpu.sync_copy(gi_hbm.at[pl.ds(off, ge)], idx_v)
            cps = [
                pltpu.async_copy(dflat.at[idx_v.at[pl.ds(b * 128, 128)]],
                                 val_v.at[pl.ds(b * 128, 128)], sem)
                for b in range(grp_n)
            ]
            for cp in cps:
                cp.wait()
            pltpu.sync_copy(val_v, out_hbm.at[pl.ds(off, ge)])
            return carry

        lax.fori_loop(0, groups, grp, 0)

    mesh = plsc.VectorSubcoreMesh(core_axis_name="c", subcore_axis_name="s")
    return pl.kernel(
        body,
        out_type=jax.ShapeDtypeStruct((n_idx,), jnp.float32),
        mesh=mesh,
        scratch_types=[
            pltpu.VMEM((ge,), jnp.int32),
            pltpu.VMEM((ge,), jnp.float32),
            pltpu.SemaphoreType.DMA,
        ],
    )(d2d, giflat)


def _loss_body(g1_ref, g2_ref, tv1_ref, tv2_ref, sw_ref, out_ref):
    cid = lax.broadcasted_iota(jnp.int32, (B, GW), 1)
    colmask = cid < VALID
    negmask = colmask & (cid >= P)
    c = K * (1.0 / ND)
    sw = sw_ref[...]

    def side_loss(g, tv):
        x = jnp.where(colmask, jnp.exp(g * (1.0 / T)), 0.0)
        z = (jnp.sum(x) / (B * VALID)) * ND
        out = x / z
        ppos = out[:, 0:P]
        log_d1 = jnp.log(ppos / (ppos + c + EPS))
        log_d0 = jnp.where(negmask, jnp.log(c / (out + c + EPS)), 0.0)
        s0 = jnp.sum(log_d0, axis=1, keepdims=True)
        ks = tv[:, 0:P]
        num = jnp.sum((log_d1 + s0) * ks, axis=1, keepdims=True)
        den = jnp.sum(ks, axis=1, keepdims=True)
        return jnp.sum(-(num / den) * sw, axis=(0, 1), keepdims=True) / B

    loss_s = side_loss(g2_ref[...], tv2_ref[...])
    loss_t = side_loss(g1_ref[...], tv1_ref[...])
    out_ref[...] = loss_s + loss_t


def _loss(g1, g2, tv1, tv2, sw):
    return pl.pallas_call(
        _loss_body,
        out_shape=jax.ShapeDtypeStruct((1, 1), jnp.float32),
    )(g1, g2, tv1, tv2, sw)


def kernel(f_s, f_t, sample_weights, idx, batch_label, num_pos,
           W_s, b_s, W_t, b_t, memory_v1, memory_v2):
    idx = idx.astype(jnp.int32)
    qr1 = memory_v1[idx[:, 0]]
    qr2 = memory_v2[idx[:, 0]]
    v1, v2, q1n, q2n = _embed(f_s, f_t, W_s, b_s, W_t, b_t, qr1, qr2)
    mem1p = jnp.pad(memory_v1, ((0, NDP - ND), (0, 0)))
    mem2p = jnp.pad(memory_v2, ((0, NDP - ND), (0, 0)))
    lbl2 = batch_label.astype(jnp.int32).reshape(B, 1)
    nchunk = 2
    h = B // nchunk
    chunks = []
    for qn, v, memp in ((q1n, v2, mem1p), (q2n, v1, mem2p)):
        for r0 in range(0, B, h):
            d, tv, ti = _simdots(qn[r0:r0 + h], v[r0:r0 + h],
                                 lbl2[r0:r0 + h], memp)
            gi = _prep(idx[r0:r0 + h], ti)
            g = _gather_sc(d.reshape(-1), gi.reshape(-1)).reshape(h, GW)
            chunks.append((g, tv))
    g1 = jnp.concatenate([c[0] for c in chunks[:nchunk]], axis=0)
    tv1 = jnp.concatenate([c[1] for c in chunks[:nchunk]], axis=0)
    g2 = jnp.concatenate([c[0] for c in chunks[nchunk:]], axis=0)
    tv2 = jnp.concatenate([c[1] for c in chunks[nchunk:]], axis=0)
    loss = _loss(g1, g2, tv1, tv2, sample_weights.reshape(B, 1))
    return loss.reshape(())
